# Initial kernel scaffold; baseline (speedup 1.0000x reference)
import jax
import jax.numpy as jnp
from jax.experimental import pallas as pl


def kernel(ego_feats, col_feats, W1, g1, b1, W2, g2, b2, ego_lin, col_lin):
    raise NotImplementedError("write your pallas kernel here")


# speedup vs baseline: 2.9568x; 2.9568x over previous
import jax
import jax.numpy as jnp
from jax.experimental import pallas as pl

B_, H_, W_ = 4, 256, 256
HW_ = H_ * W_
TOTAL_ = B_ * HW_
C_IN_, C1_, C2_ = 128, 96, 64
BR = 8


def _conv_block(cat3, maskc, Wk, r0, c_out):
    cin = cat3.shape[-1]
    rows = r0 + jax.lax.broadcasted_iota(jnp.int32, (BR, 1, 1), 0)
    y = rows % H_
    acc = jnp.zeros((BR * W_, c_out), jnp.float32)
    for dy in (-1, 0, 1):
        src = cat3[BR + dy:2 * BR + dy]
        valid = ((y + dy >= 0) & (y + dy < H_)).astype(jnp.float32)
        src = src * valid
        for dx in (-1, 0, 1):
            if dx == -1:
                sh = jnp.concatenate(
                    [jnp.zeros((BR, 1, cin), jnp.float32), src[:, :-1, :]], axis=1)
            elif dx == 1:
                sh = jnp.concatenate(
                    [src[:, 1:, :], jnp.zeros((BR, 1, cin), jnp.float32)], axis=1)
            else:
                sh = src
            acc = acc + jnp.dot(sh.reshape(BR * W_, cin), Wk[dy + 1, dx + 1],
                                preferred_element_type=jnp.float32)
    hm = acc.reshape(BR, W_, c_out) * maskc[:, :, None]
    s1 = hm.sum(axis=(0, 1))
    s2 = (hm * hm).sum(axis=(0, 1))
    return hm, s1, s2


def _k1(gm, gc, gp, mc, w1, out, s1o, s2o):
    cat3 = jnp.concatenate([gm[...], gc[...], gp[...]], axis=0)
    r0 = pl.program_id(0) * BR
    hm, s1, s2 = _conv_block(cat3, mc[...], w1[...], r0, C1_)
    out[...] = hm
    s1o[...] = s1.reshape(1, 1, C1_)
    s2o[...] = s2.reshape(1, 1, C1_)


def _k2(hm_, hc_, hp_, mm_, mc_, mp_, ge, be, w2, out, s1o, s2o):
    cat3 = jnp.concatenate([hm_[...], hc_[...], hp_[...]], axis=0)
    catm = jnp.concatenate([mm_[...], mc_[...], mp_[...]], axis=0)
    g = ge[...].reshape(1, 1, C1_)
    b = be[...].reshape(1, 1, C1_)
    catn = jax.nn.relu(cat3 * g + b) * catm[:, :, None]
    r0 = pl.program_id(0) * BR
    hm, s1, s2 = _conv_block(catn, mc_[...], w2[...], r0, C2_)
    out[...] = hm
    s1o[...] = s1.reshape(1, 1, C2_)
    s2o[...] = s2.reshape(1, 1, C2_)


def _k3(x, ge, be, out):
    out[...] = jax.nn.relu(x[...] * ge[...].reshape(1, C2_) + be[...].reshape(1, C2_))


def _affine(s1, s2, count, gamma, beta):
    mean = s1 / count
    var = s2 / count - mean * mean
    inv = gamma / jnp.sqrt(var + 1e-5)
    return inv, beta - mean * inv


def kernel(ego_feats, col_feats, W1, g1, b1, W2, g2, b2, ego_lin, col_lin):
    nb = (B_ * H_) // BR
    ego_pad = jnp.concatenate([ego_feats, jnp.zeros_like(ego_feats)], axis=1)
    col_pad = jnp.concatenate([jnp.zeros_like(col_feats), col_feats], axis=1)
    grid = (jnp.zeros((TOTAL_, C_IN_), jnp.float32)
            .at[ego_lin].add(ego_pad).at[col_lin].add(col_pad))
    mask = (jnp.zeros((TOTAL_,), jnp.float32)
            .at[ego_lin].set(1.0).at[col_lin].set(1.0))
    count = mask.sum()

    grid3 = grid.reshape(B_ * H_, W_, C_IN_)
    mask2 = mask.reshape(B_ * H_, W_)
    gpad = jnp.pad(grid3, ((BR, BR), (0, 0), (0, 0)))
    mpad = jnp.pad(mask2, ((BR, BR), (0, 0)))

    shift_specs = lambda c: [
        pl.BlockSpec((BR, W_, c), lambda i: (i, 0, 0)),
        pl.BlockSpec((BR, W_, c), lambda i: (i + 1, 0, 0)),
        pl.BlockSpec((BR, W_, c), lambda i: (i + 2, 0, 0)),
    ]
    mshift_specs = [
        pl.BlockSpec((BR, W_), lambda i: (i, 0)),
        pl.BlockSpec((BR, W_), lambda i: (i + 1, 0)),
        pl.BlockSpec((BR, W_), lambda i: (i + 2, 0)),
    ]
    stat_spec = lambda c: pl.BlockSpec((1, 1, c), lambda i: (i, 0, 0))
    full = lambda shape: pl.BlockSpec(shape, lambda i: tuple(0 for _ in shape))

    h1, s11, s12 = pl.pallas_call(
        _k1,
        grid=(nb,),
        in_specs=shift_specs(C_IN_) + [
            pl.BlockSpec((BR, W_), lambda i: (i, 0)),
            full((3, 3, C_IN_, C1_)),
        ],
        out_specs=[
            pl.BlockSpec((BR, W_, C1_), lambda i: (i, 0, 0)),
            stat_spec(C1_), stat_spec(C1_),
        ],
        out_shape=[
            jax.ShapeDtypeStruct((B_ * H_, W_, C1_), jnp.float32),
            jax.ShapeDtypeStruct((nb, 1, C1_), jnp.float32),
            jax.ShapeDtypeStruct((nb, 1, C1_), jnp.float32),
        ],
    )(gpad, gpad, gpad, mask2, W1)

    ge1, be1 = _affine(s11.sum(axis=(0, 1)), s12.sum(axis=(0, 1)), count, g1, b1)

    h1pad = jnp.pad(h1, ((BR, BR), (0, 0), (0, 0)))
    h2, s21, s22 = pl.pallas_call(
        _k2,
        grid=(nb,),
        in_specs=shift_specs(C1_) + mshift_specs + [
            full((1, C1_)), full((1, C1_)), full((3, 3, C1_, C2_)),
        ],
        out_specs=[
            pl.BlockSpec((BR, W_, C2_), lambda i: (i, 0, 0)),
            stat_spec(C2_), stat_spec(C2_),
        ],
        out_shape=[
            jax.ShapeDtypeStruct((B_ * H_, W_, C2_), jnp.float32),
            jax.ShapeDtypeStruct((nb, 1, C2_), jnp.float32),
            jax.ShapeDtypeStruct((nb, 1, C2_), jnp.float32),
        ],
    )(h1pad, h1pad, h1pad, mpad, mpad, mpad,
      ge1.reshape(1, C1_), be1.reshape(1, C1_), W2)

    ge2, be2 = _affine(s21.sum(axis=(0, 1)), s22.sum(axis=(0, 1)), count, g2, b2)

    active = jnp.concatenate([ego_lin, col_lin]).astype(jnp.int32)
    n_act = active.shape[0]
    blk = 2048
    n_pad = ((n_act + blk - 1) // blk) * blk
    act_p = jnp.pad(active, (0, n_pad - n_act))
    h2g = h2.reshape(TOTAL_, C2_)[act_p]

    out = pl.pallas_call(
        _k3,
        grid=(n_pad // blk,),
        in_specs=[
            pl.BlockSpec((blk, C2_), lambda i: (i, 0)),
            full((1, C2_)), full((1, C2_)),
        ],
        out_specs=pl.BlockSpec((blk, C2_), lambda i: (i, 0)),
        out_shape=jax.ShapeDtypeStruct((n_pad, C2_), jnp.float32),
    )(h2g, ge2.reshape(1, C2_), be2.reshape(1, C2_))
    return out[:n_act]
